# baseline (device time: 765582 ns/iter reference)
import jax
import jax.numpy as jnp
from jax import lax
from jax.experimental import pallas as pl
from jax.experimental.pallas import tpu as pltpu

N_DEV = 16


def kernel(x, w_mat):
    m_per, k = x.shape
    k2, n_per = w_mat.shape
    assert k == k2

    def body(x_ref, w_ref, out_ref, comm_ref, send_sems, recv_sems, credit_sem):
        my = lax.axis_index("i")
        left = jax.lax.rem(my - 1 + N_DEV, N_DEV)
        right = jax.lax.rem(my + 1, N_DEV)

        barrier_sem = pltpu.get_barrier_semaphore()
        for nbr in (left, right):
            pl.semaphore_signal(
                barrier_sem, inc=1,
                device_id=(nbr,), device_id_type=pl.DeviceIdType.MESH,
            )
        pl.semaphore_wait(barrier_sem, 2)

        comm_ref[0, :, :] = x_ref[:, :]
        out_ref[pl.ds(my * m_per, m_per), :] = jnp.dot(
            x_ref[:, :], w_ref[:, :], preferred_element_type=jnp.float32
        )

        for h in range(N_DEV - 1):
            s = h % 2
            r = (h + 1) % 2
            if h >= 1:
                pl.semaphore_wait(credit_sem, 1)
            rdma = pltpu.make_async_remote_copy(
                src_ref=comm_ref.at[s],
                dst_ref=comm_ref.at[r],
                send_sem=send_sems.at[s],
                recv_sem=recv_sems.at[r],
                device_id=(right,),
                device_id_type=pl.DeviceIdType.MESH,
            )
            rdma.start()
            rdma.wait()
            if h < N_DEV - 2:
                pl.semaphore_signal(
                    credit_sem, inc=1,
                    device_id=(left,), device_id_type=pl.DeviceIdType.MESH,
                )
            origin = jax.lax.rem(my - h - 1 + N_DEV, N_DEV)
            out_ref[pl.ds(origin * m_per, m_per), :] = jnp.dot(
                comm_ref[r], w_ref[:, :], preferred_element_type=jnp.float32
            )

    return pl.pallas_call(
        body,
        out_shape=jax.ShapeDtypeStruct((N_DEV * m_per, n_per), jnp.float32),
        in_specs=[
            pl.BlockSpec(memory_space=pltpu.VMEM),
            pl.BlockSpec(memory_space=pltpu.VMEM),
        ],
        out_specs=pl.BlockSpec(memory_space=pltpu.VMEM),
        scratch_shapes=[
            pltpu.VMEM((2, m_per, k), jnp.float32),
            pltpu.SemaphoreType.DMA((2,)),
            pltpu.SemaphoreType.DMA((2,)),
            pltpu.SemaphoreType.REGULAR,
        ],
        compiler_params=pltpu.CompilerParams(collective_id=0),
    )(x, w_mat)


# device time: 391718 ns/iter; 1.9544x vs baseline; 1.9544x over previous
import jax
import jax.numpy as jnp
from jax import lax
from jax.experimental import pallas as pl
from jax.experimental.pallas import tpu as pltpu

N_DEV = 16
N_HOP = 8


def kernel(x, w_mat):
    m_per, k = x.shape
    k2, n_per = w_mat.shape
    assert k == k2
    half = m_per // 2

    def body(x_ref, w_ref, out_ref,
             cw_comm, ccw_comm,
             cw_send_sems, cw_recv_sems, ccw_send_sems, ccw_recv_sems,
             cw_credit, ccw_credit):
        my = lax.axis_index("i")
        left = jax.lax.rem(my - 1 + N_DEV, N_DEV)
        right = jax.lax.rem(my + 1, N_DEV)

        barrier_sem = pltpu.get_barrier_semaphore()
        for nbr in (left, right):
            pl.semaphore_signal(
                barrier_sem, inc=1,
                device_id=(nbr,), device_id_type=pl.DeviceIdType.MESH,
            )
        pl.semaphore_wait(barrier_sem, 2)

        cw_comm[0, :, :] = x_ref[:, :]
        ccw_comm[0, :, :] = x_ref[:, :]

        def gemm(rows_ref, origin, row_off):
            out_ref[pl.ds(origin * m_per + row_off, rows_ref.shape[0]), :] = (
                jnp.dot(rows_ref[:, :], w_ref[:, :],
                        preferred_element_type=jnp.float32)
            )

        for h in range(N_HOP):
            s = h % 2
            r = (h + 1) % 2
            if h >= 1:
                pl.semaphore_wait(cw_credit, 1)
            if h == N_HOP - 1:
                cw_rdma = pltpu.make_async_remote_copy(
                    src_ref=cw_comm.at[s, pl.ds(0, half)],
                    dst_ref=cw_comm.at[r, pl.ds(0, half)],
                    send_sem=cw_send_sems.at[s],
                    recv_sem=cw_recv_sems.at[r],
                    device_id=(right,),
                    device_id_type=pl.DeviceIdType.MESH,
                )
            else:
                cw_rdma = pltpu.make_async_remote_copy(
                    src_ref=cw_comm.at[s],
                    dst_ref=cw_comm.at[r],
                    send_sem=cw_send_sems.at[s],
                    recv_sem=cw_recv_sems.at[r],
                    device_id=(right,),
                    device_id_type=pl.DeviceIdType.MESH,
                )
            cw_rdma.start()
            if h >= 1:
                pl.semaphore_wait(ccw_credit, 1)
            if h == N_HOP - 1:
                ccw_rdma = pltpu.make_async_remote_copy(
                    src_ref=ccw_comm.at[s, pl.ds(half, half)],
                    dst_ref=ccw_comm.at[r, pl.ds(half, half)],
                    send_sem=ccw_send_sems.at[s],
                    recv_sem=ccw_recv_sems.at[r],
                    device_id=(left,),
                    device_id_type=pl.DeviceIdType.MESH,
                )
            else:
                ccw_rdma = pltpu.make_async_remote_copy(
                    src_ref=ccw_comm.at[s],
                    dst_ref=ccw_comm.at[r],
                    send_sem=ccw_send_sems.at[s],
                    recv_sem=ccw_recv_sems.at[r],
                    device_id=(left,),
                    device_id_type=pl.DeviceIdType.MESH,
                )
            ccw_rdma.start()

            if h == 0:
                gemm(x_ref, my, 0)
            else:
                gemm(cw_comm.at[s], jax.lax.rem(my - h + N_DEV, N_DEV), 0)
                gemm(ccw_comm.at[s], jax.lax.rem(my + h, N_DEV), 0)

            cw_rdma.wait()
            ccw_rdma.wait()
            if h < N_HOP - 1:
                pl.semaphore_signal(
                    cw_credit, inc=1,
                    device_id=(left,), device_id_type=pl.DeviceIdType.MESH,
                )
                pl.semaphore_signal(
                    ccw_credit, inc=1,
                    device_id=(right,), device_id_type=pl.DeviceIdType.MESH,
                )

        r_last = N_HOP % 2
        origin8 = jax.lax.rem(my + N_HOP, N_DEV)
        gemm(cw_comm.at[r_last, pl.ds(0, half)], origin8, 0)
        gemm(ccw_comm.at[r_last, pl.ds(half, half)], origin8, half)

    return pl.pallas_call(
        body,
        out_shape=jax.ShapeDtypeStruct((N_DEV * m_per, n_per), jnp.float32),
        in_specs=[
            pl.BlockSpec(memory_space=pltpu.VMEM),
            pl.BlockSpec(memory_space=pltpu.VMEM),
        ],
        out_specs=pl.BlockSpec(memory_space=pltpu.VMEM),
        scratch_shapes=[
            pltpu.VMEM((2, m_per, k), jnp.float32),
            pltpu.VMEM((2, m_per, k), jnp.float32),
            pltpu.SemaphoreType.DMA((2,)),
            pltpu.SemaphoreType.DMA((2,)),
            pltpu.SemaphoreType.DMA((2,)),
            pltpu.SemaphoreType.DMA((2,)),
            pltpu.SemaphoreType.REGULAR,
            pltpu.SemaphoreType.REGULAR,
        ],
        compiler_params=pltpu.CompilerParams(collective_id=0),
    )(x, w_mat)


# device time: 385665 ns/iter; 1.9851x vs baseline; 1.0157x over previous
import jax
import jax.numpy as jnp
from jax import lax
from jax.experimental import pallas as pl
from jax.experimental.pallas import tpu as pltpu

N_DEV = 16
N_HOP = 8

RING = [0, 1, 5, 9, 13, 14, 10, 6, 2, 3, 7, 11, 15, 12, 8, 4]
POS = [RING.index(i) for i in range(N_DEV)]


def kernel(x, w_mat):
    m_per, k = x.shape
    k2, n_per = w_mat.shape
    assert k == k2
    half = m_per // 2

    my = lax.axis_index("i")
    ring = jnp.array(RING, dtype=jnp.int32)
    my_pos = jnp.array(POS, dtype=jnp.int32)[my]
    ids = jnp.concatenate([
        jnp.stack([
            ring[(my_pos - 1) % N_DEV],
            ring[(my_pos + 1) % N_DEV],
            my.astype(jnp.int32),
        ]),
        jnp.stack([ring[(my_pos - h) % N_DEV] for h in range(1, N_HOP)]),
        jnp.stack([ring[(my_pos + h) % N_DEV] for h in range(1, N_HOP)]),
        jnp.stack([ring[(my_pos + N_HOP) % N_DEV]]),
    ])

    def body(ids_ref, x_ref, w_ref, out_ref,
             cw_comm, ccw_comm,
             cw_send_sems, cw_recv_sems, ccw_send_sems, ccw_recv_sems,
             cw_credit, ccw_credit):
        left = ids_ref[0]
        right = ids_ref[1]
        my = ids_ref[2]

        barrier_sem = pltpu.get_barrier_semaphore()
        for nbr in (left, right):
            pl.semaphore_signal(
                barrier_sem, inc=1,
                device_id=(nbr,), device_id_type=pl.DeviceIdType.MESH,
            )
        pl.semaphore_wait(barrier_sem, 2)

        cw_comm[0, :, :] = x_ref[:, :]
        ccw_comm[0, :, :] = x_ref[:, :]

        def gemm(rows_ref, origin, row_off):
            out_ref[pl.ds(origin * m_per + row_off, rows_ref.shape[0]), :] = (
                jnp.dot(rows_ref[:, :], w_ref[:, :],
                        preferred_element_type=jnp.float32)
            )

        for h in range(N_HOP):
            s = h % 2
            r = (h + 1) % 2
            if h >= 1:
                pl.semaphore_wait(cw_credit, 1)
            if h == N_HOP - 1:
                cw_rdma = pltpu.make_async_remote_copy(
                    src_ref=cw_comm.at[s, pl.ds(0, half)],
                    dst_ref=cw_comm.at[r, pl.ds(0, half)],
                    send_sem=cw_send_sems.at[s],
                    recv_sem=cw_recv_sems.at[r],
                    device_id=(right,),
                    device_id_type=pl.DeviceIdType.MESH,
                )
            else:
                cw_rdma = pltpu.make_async_remote_copy(
                    src_ref=cw_comm.at[s],
                    dst_ref=cw_comm.at[r],
                    send_sem=cw_send_sems.at[s],
                    recv_sem=cw_recv_sems.at[r],
                    device_id=(right,),
                    device_id_type=pl.DeviceIdType.MESH,
                )
            cw_rdma.start()
            if h >= 1:
                pl.semaphore_wait(ccw_credit, 1)
            if h == N_HOP - 1:
                ccw_rdma = pltpu.make_async_remote_copy(
                    src_ref=ccw_comm.at[s, pl.ds(half, half)],
                    dst_ref=ccw_comm.at[r, pl.ds(half, half)],
                    send_sem=ccw_send_sems.at[s],
                    recv_sem=ccw_recv_sems.at[r],
                    device_id=(left,),
                    device_id_type=pl.DeviceIdType.MESH,
                )
            else:
                ccw_rdma = pltpu.make_async_remote_copy(
                    src_ref=ccw_comm.at[s],
                    dst_ref=ccw_comm.at[r],
                    send_sem=ccw_send_sems.at[s],
                    recv_sem=ccw_recv_sems.at[r],
                    device_id=(left,),
                    device_id_type=pl.DeviceIdType.MESH,
                )
            ccw_rdma.start()

            if h == 0:
                gemm(x_ref, my, 0)
            else:
                gemm(cw_comm.at[s], ids_ref[3 + (h - 1)], 0)
                gemm(ccw_comm.at[s], ids_ref[3 + (N_HOP - 1) + (h - 1)], 0)

            cw_rdma.wait()
            ccw_rdma.wait()
            if h < N_HOP - 1:
                pl.semaphore_signal(
                    cw_credit, inc=1,
                    device_id=(left,), device_id_type=pl.DeviceIdType.MESH,
                )
                pl.semaphore_signal(
                    ccw_credit, inc=1,
                    device_id=(right,), device_id_type=pl.DeviceIdType.MESH,
                )

        r_last = N_HOP % 2
        origin8 = ids_ref[3 + 2 * (N_HOP - 1)]
        gemm(cw_comm.at[r_last, pl.ds(0, half)], origin8, 0)
        gemm(ccw_comm.at[r_last, pl.ds(half, half)], origin8, half)

    return pl.pallas_call(
        body,
        out_shape=jax.ShapeDtypeStruct((N_DEV * m_per, n_per), jnp.float32),
        in_specs=[
            pl.BlockSpec(memory_space=pltpu.SMEM),
            pl.BlockSpec(memory_space=pltpu.VMEM),
            pl.BlockSpec(memory_space=pltpu.VMEM),
        ],
        out_specs=pl.BlockSpec(memory_space=pltpu.VMEM),
        scratch_shapes=[
            pltpu.VMEM((2, m_per, k), jnp.float32),
            pltpu.VMEM((2, m_per, k), jnp.float32),
            pltpu.SemaphoreType.DMA((2,)),
            pltpu.SemaphoreType.DMA((2,)),
            pltpu.SemaphoreType.DMA((2,)),
            pltpu.SemaphoreType.DMA((2,)),
            pltpu.SemaphoreType.REGULAR,
            pltpu.SemaphoreType.REGULAR,
        ],
        compiler_params=pltpu.CompilerParams(collective_id=0),
    )(ids, x, w_mat)


# device time: 375166 ns/iter; 2.0406x vs baseline; 1.0280x over previous
import jax
import jax.numpy as jnp
from jax import lax
from jax.experimental import pallas as pl
from jax.experimental.pallas import tpu as pltpu

N_DEV = 16
N_HOP = 8
N_SLOT = 3
N_SUB = 2

RING = [0, 1, 5, 9, 13, 14, 10, 6, 2, 3, 7, 11, 15, 12, 8, 4]
POS = [RING.index(i) for i in range(N_DEV)]


def kernel(x, w_mat):
    m_per, k = x.shape
    k2, n_per = w_mat.shape
    assert k == k2
    sub = m_per // N_SUB

    my = lax.axis_index("i")
    ring = jnp.array(RING, dtype=jnp.int32)
    my_pos = jnp.array(POS, dtype=jnp.int32)[my]
    ids = jnp.concatenate([
        jnp.stack([
            ring[(my_pos - 1) % N_DEV],
            ring[(my_pos + 1) % N_DEV],
            my.astype(jnp.int32),
        ]),
        jnp.stack([ring[(my_pos - h) % N_DEV] for h in range(1, N_HOP)]),
        jnp.stack([ring[(my_pos + h) % N_DEV] for h in range(1, N_HOP)]),
        jnp.stack([ring[(my_pos + N_HOP) % N_DEV]]),
    ])

    def body(ids_ref, x_ref, w_ref, out_ref,
             cw_comm, ccw_comm,
             cw_send_sems, cw_recv_sems, ccw_send_sems, ccw_recv_sems,
             cw_credit, ccw_credit):
        left = ids_ref[0]
        right = ids_ref[1]
        my = ids_ref[2]

        barrier_sem = pltpu.get_barrier_semaphore()
        for nbr in (left, right):
            pl.semaphore_signal(
                barrier_sem, inc=1,
                device_id=(nbr,), device_id_type=pl.DeviceIdType.MESH,
            )
        pl.semaphore_wait(barrier_sem, 2)

        cw_comm[0, :, :] = x_ref[:, :]
        ccw_comm[0, :, :] = x_ref[:, :]

        def gemm(rows_ref, origin, row_off):
            out_ref[pl.ds(origin * m_per + row_off, rows_ref.shape[0]), :] = (
                jnp.dot(rows_ref[:, :], w_ref[:, :],
                        preferred_element_type=jnp.float32)
            )

        def make_copy(comm, send_sems, recv_sems, src_slot, dst_slot, i, tgt):
            return pltpu.make_async_remote_copy(
                src_ref=comm.at[src_slot, pl.ds(i * sub, sub)],
                dst_ref=comm.at[dst_slot, pl.ds(i * sub, sub)],
                send_sem=send_sems.at[src_slot, i],
                recv_sem=recv_sems.at[dst_slot, i],
                device_id=(tgt,),
                device_id_type=pl.DeviceIdType.MESH,
            )

        prev_cw = None
        prev_ccw = None
        for h in range(N_HOP):
            src_slot = h % N_SLOT
            dst_slot = (h + 1) % N_SLOT
            if h >= N_SLOT - 1:
                pl.semaphore_wait(cw_credit, 1)
                pl.semaphore_wait(ccw_credit, 1)

            cw_subs = [0] if h == N_HOP - 1 else [0, 1]
            ccw_subs = [1] if h == N_HOP - 1 else [0, 1]
            cur_cw, cur_ccw = {}, {}
            for i in cw_subs:
                if h >= 1:
                    prev_cw[i].wait_recv()
                cur_cw[i] = make_copy(cw_comm, cw_send_sems, cw_recv_sems,
                                      src_slot, dst_slot, i, right)
                cur_cw[i].start()
            for i in ccw_subs:
                if h >= 1:
                    prev_ccw[i].wait_recv()
                cur_ccw[i] = make_copy(ccw_comm, ccw_send_sems, ccw_recv_sems,
                                       src_slot, dst_slot, i, left)
                cur_ccw[i].start()

            if h == 0:
                gemm(x_ref, my, 0)
            else:
                if h == N_HOP - 1:
                    prev_cw[1].wait_recv()
                    prev_ccw[0].wait_recv()
                gemm(cw_comm.at[src_slot], ids_ref[3 + (h - 1)], 0)
                gemm(ccw_comm.at[src_slot],
                     ids_ref[3 + (N_HOP - 1) + (h - 1)], 0)

            if h <= N_HOP - N_SLOT:
                for i in (0, 1):
                    cur_cw[i].wait_send()
                    cur_ccw[i].wait_send()
                pl.semaphore_signal(
                    cw_credit, inc=1,
                    device_id=(left,), device_id_type=pl.DeviceIdType.MESH,
                )
                pl.semaphore_signal(
                    ccw_credit, inc=1,
                    device_id=(right,), device_id_type=pl.DeviceIdType.MESH,
                )
            elif h == N_HOP - 1:
                for i in (0, 1):
                    prev_cw[i].wait_send()
                    prev_ccw[i].wait_send()
            prev_cw, prev_ccw = cur_cw, cur_ccw

        last_slot = N_HOP % N_SLOT
        origin8 = ids_ref[3 + 2 * (N_HOP - 1)]
        prev_cw[0].wait_recv()
        gemm(cw_comm.at[last_slot, pl.ds(0, sub)], origin8, 0)
        prev_ccw[1].wait_recv()
        gemm(ccw_comm.at[last_slot, pl.ds(sub, sub)], origin8, sub)
        prev_cw[0].wait_send()
        prev_ccw[1].wait_send()

    return pl.pallas_call(
        body,
        out_shape=jax.ShapeDtypeStruct((N_DEV * m_per, n_per), jnp.float32),
        in_specs=[
            pl.BlockSpec(memory_space=pltpu.SMEM),
            pl.BlockSpec(memory_space=pltpu.VMEM),
            pl.BlockSpec(memory_space=pltpu.VMEM),
        ],
        out_specs=pl.BlockSpec(memory_space=pltpu.VMEM),
        scratch_shapes=[
            pltpu.VMEM((N_SLOT, m_per, k), jnp.float32),
            pltpu.VMEM((N_SLOT, m_per, k), jnp.float32),
            pltpu.SemaphoreType.DMA((N_SLOT, N_SUB)),
            pltpu.SemaphoreType.DMA((N_SLOT, N_SUB)),
            pltpu.SemaphoreType.DMA((N_SLOT, N_SUB)),
            pltpu.SemaphoreType.DMA((N_SLOT, N_SUB)),
            pltpu.SemaphoreType.REGULAR,
            pltpu.SemaphoreType.REGULAR,
        ],
        compiler_params=pltpu.CompilerParams(collective_id=0),
    )(ids, x, w_mat)


# device time: 369578 ns/iter; 2.0715x vs baseline; 1.0151x over previous
import jax
import jax.numpy as jnp
from jax import lax
from jax.experimental import pallas as pl
from jax.experimental.pallas import tpu as pltpu

N_DEV = 16
N_HOP = 8
N_SLOT = 4
N_SUB = 2

RING = [0, 1, 5, 9, 13, 14, 10, 6, 2, 3, 7, 11, 15, 12, 8, 4]
POS = [RING.index(i) for i in range(N_DEV)]


def kernel(x, w_mat):
    m_per, k = x.shape
    k2, n_per = w_mat.shape
    assert k == k2
    sub = m_per // N_SUB

    my = lax.axis_index("i")
    ring = jnp.array(RING, dtype=jnp.int32)
    my_pos = jnp.array(POS, dtype=jnp.int32)[my]
    ids = jnp.concatenate([
        jnp.stack([
            ring[(my_pos - 1) % N_DEV],
            ring[(my_pos + 1) % N_DEV],
            my.astype(jnp.int32),
        ]),
        jnp.stack([ring[(my_pos - h) % N_DEV] for h in range(1, N_HOP)]),
        jnp.stack([ring[(my_pos + h) % N_DEV] for h in range(1, N_HOP)]),
        jnp.stack([ring[(my_pos + N_HOP) % N_DEV]]),
    ])

    def body(ids_ref, x_ref, w_ref, out_ref,
             cw_comm, ccw_comm,
             cw_send_sems, cw_recv_sems, ccw_send_sems, ccw_recv_sems,
             cw_credit, ccw_credit):
        left = ids_ref[0]
        right = ids_ref[1]
        my = ids_ref[2]

        barrier_sem = pltpu.get_barrier_semaphore()
        for nbr in (left, right):
            pl.semaphore_signal(
                barrier_sem, inc=1,
                device_id=(nbr,), device_id_type=pl.DeviceIdType.MESH,
            )
        pl.semaphore_wait(barrier_sem, 2)

        def gemm(rows_ref, origin, row_off):
            out_ref[pl.ds(origin * m_per + row_off, rows_ref.shape[0]), :] = (
                jnp.dot(rows_ref[:, :], w_ref[:, :],
                        preferred_element_type=jnp.float32)
            )

        def make_copy(comm, send_sems, recv_sems, src_slot, dst_slot, i, tgt):
            src = (x_ref if src_slot is None else comm.at[src_slot])
            sem_slot = 0 if src_slot is None else src_slot
            return pltpu.make_async_remote_copy(
                src_ref=src.at[pl.ds(i * sub, sub)],
                dst_ref=comm.at[dst_slot, pl.ds(i * sub, sub)],
                send_sem=send_sems.at[sem_slot, i],
                recv_sem=recv_sems.at[dst_slot, i],
                device_id=(tgt,),
                device_id_type=pl.DeviceIdType.MESH,
            )

        prev_cw = None
        prev_ccw = None
        for h in range(N_HOP):
            src_slot = None if h == 0 else h % N_SLOT
            dst_slot = (h + 1) % N_SLOT
            if h >= N_SLOT:
                pl.semaphore_wait(cw_credit, 1)
                pl.semaphore_wait(ccw_credit, 1)

            cw_subs = [0] if h == N_HOP - 1 else [0, 1]
            ccw_subs = [1] if h == N_HOP - 1 else [0, 1]
            cur_cw, cur_ccw = {}, {}
            for i in cw_subs:
                if h >= 1:
                    prev_cw[i].wait_recv()
                cur_cw[i] = make_copy(cw_comm, cw_send_sems, cw_recv_sems,
                                      src_slot, dst_slot, i, right)
                cur_cw[i].start()
            for i in ccw_subs:
                if h >= 1:
                    prev_ccw[i].wait_recv()
                cur_ccw[i] = make_copy(ccw_comm, ccw_send_sems, ccw_recv_sems,
                                       src_slot, dst_slot, i, left)
                cur_ccw[i].start()

            if h == 0:
                gemm(x_ref, my, 0)
            else:
                if h == N_HOP - 1:
                    prev_cw[1].wait_recv()
                    prev_ccw[0].wait_recv()
                gemm(cw_comm.at[src_slot], ids_ref[3 + (h - 1)], 0)
                gemm(ccw_comm.at[src_slot],
                     ids_ref[3 + (N_HOP - 1) + (h - 1)], 0)

            if h >= 1:
                for i in (0, 1):
                    prev_cw[i].wait_send()
                    prev_ccw[i].wait_send()
                if 1 <= h - 1 <= N_HOP - N_SLOT:
                    pl.semaphore_signal(
                        cw_credit, inc=1,
                        device_id=(left,), device_id_type=pl.DeviceIdType.MESH,
                    )
                    pl.semaphore_signal(
                        ccw_credit, inc=1,
                        device_id=(right,), device_id_type=pl.DeviceIdType.MESH,
                    )
            prev_cw, prev_ccw = cur_cw, cur_ccw

        last_slot = N_HOP % N_SLOT
        origin8 = ids_ref[3 + 2 * (N_HOP - 1)]
        prev_cw[0].wait_recv()
        gemm(cw_comm.at[last_slot, pl.ds(0, sub)], origin8, 0)
        prev_ccw[1].wait_recv()
        gemm(ccw_comm.at[last_slot, pl.ds(sub, sub)], origin8, sub)
        prev_cw[0].wait_send()
        prev_ccw[1].wait_send()

    return pl.pallas_call(
        body,
        out_shape=jax.ShapeDtypeStruct((N_DEV * m_per, n_per), jnp.float32),
        in_specs=[
            pl.BlockSpec(memory_space=pltpu.SMEM),
            pl.BlockSpec(memory_space=pltpu.VMEM),
            pl.BlockSpec(memory_space=pltpu.VMEM),
        ],
        out_specs=pl.BlockSpec(memory_space=pltpu.VMEM),
        scratch_shapes=[
            pltpu.VMEM((N_SLOT, m_per, k), jnp.float32),
            pltpu.VMEM((N_SLOT, m_per, k), jnp.float32),
            pltpu.SemaphoreType.DMA((N_SLOT, N_SUB)),
            pltpu.SemaphoreType.DMA((N_SLOT, N_SUB)),
            pltpu.SemaphoreType.DMA((N_SLOT, N_SUB)),
            pltpu.SemaphoreType.DMA((N_SLOT, N_SUB)),
            pltpu.SemaphoreType.REGULAR,
            pltpu.SemaphoreType.REGULAR,
        ],
        compiler_params=pltpu.CompilerParams(collective_id=0),
    )(ids, x, w_mat)


# device time: 367782 ns/iter; 2.0816x vs baseline; 1.0049x over previous
import jax
import jax.numpy as jnp
from jax import lax
from jax.experimental import pallas as pl
from jax.experimental.pallas import tpu as pltpu

N_DEV = 16
N_HOP = 8
N_SLOT = 4
N_SUB = 4

RING = [0, 1, 5, 9, 13, 14, 10, 6, 2, 3, 7, 11, 15, 12, 8, 4]
POS = [RING.index(i) for i in range(N_DEV)]


def kernel(x, w_mat):
    m_per, k = x.shape
    k2, n_per = w_mat.shape
    assert k == k2
    sub = m_per // N_SUB

    my = lax.axis_index("i")
    ring = jnp.array(RING, dtype=jnp.int32)
    my_pos = jnp.array(POS, dtype=jnp.int32)[my]
    ids = jnp.concatenate([
        jnp.stack([
            ring[(my_pos - 1) % N_DEV],
            ring[(my_pos + 1) % N_DEV],
            my.astype(jnp.int32),
        ]),
        jnp.stack([ring[(my_pos - h) % N_DEV] for h in range(1, N_HOP)]),
        jnp.stack([ring[(my_pos + h) % N_DEV] for h in range(1, N_HOP)]),
        jnp.stack([ring[(my_pos + N_HOP) % N_DEV]]),
    ])

    def body(ids_ref, x_ref, w_ref, out_ref,
             cw_comm, ccw_comm,
             cw_send_sems, cw_recv_sems, ccw_send_sems, ccw_recv_sems,
             cw_credit, ccw_credit):
        left = ids_ref[0]
        right = ids_ref[1]
        my = ids_ref[2]

        barrier_sem = pltpu.get_barrier_semaphore()
        for nbr in (left, right):
            pl.semaphore_signal(
                barrier_sem, inc=1,
                device_id=(nbr,), device_id_type=pl.DeviceIdType.MESH,
            )
        pl.semaphore_wait(barrier_sem, 2)

        def gemm(rows_ref, origin, row_off):
            out_ref[pl.ds(origin * m_per + row_off, rows_ref.shape[0]), :] = (
                jnp.dot(rows_ref[:, :], w_ref[:, :],
                        preferred_element_type=jnp.float32)
            )

        def make_copy(comm, send_sems, recv_sems, src_slot, dst_slot, i, tgt):
            src = (x_ref if src_slot is None else comm.at[src_slot])
            sem_slot = 0 if src_slot is None else src_slot
            return pltpu.make_async_remote_copy(
                src_ref=src.at[pl.ds(i * sub, sub)],
                dst_ref=comm.at[dst_slot, pl.ds(i * sub, sub)],
                send_sem=send_sems.at[sem_slot, i],
                recv_sem=recv_sems.at[dst_slot, i],
                device_id=(tgt,),
                device_id_type=pl.DeviceIdType.MESH,
            )

        prev_cw = None
        prev_ccw = None
        for h in range(N_HOP):
            src_slot = None if h == 0 else h % N_SLOT
            dst_slot = (h + 1) % N_SLOT
            if h >= N_SLOT:
                pl.semaphore_wait(cw_credit, 1)
                pl.semaphore_wait(ccw_credit, 1)

            if h == N_HOP - 1:
                cw_subs = list(range(N_SUB // 2))
                ccw_subs = list(range(N_SUB // 2, N_SUB))
            else:
                cw_subs = ccw_subs = list(range(N_SUB))
            cur_cw, cur_ccw = {}, {}
            for i in cw_subs:
                if h >= 1:
                    prev_cw[i].wait_recv()
                cur_cw[i] = make_copy(cw_comm, cw_send_sems, cw_recv_sems,
                                      src_slot, dst_slot, i, right)
                cur_cw[i].start()
            for i in ccw_subs:
                if h >= 1:
                    prev_ccw[i].wait_recv()
                cur_ccw[i] = make_copy(ccw_comm, ccw_send_sems, ccw_recv_sems,
                                       src_slot, dst_slot, i, left)
                cur_ccw[i].start()

            if h == 0:
                gemm(x_ref, my, 0)
            else:
                if h == N_HOP - 1:
                    for i in range(N_SUB // 2, N_SUB):
                        prev_cw[i].wait_recv()
                    for i in range(N_SUB // 2):
                        prev_ccw[i].wait_recv()
                gemm(cw_comm.at[src_slot], ids_ref[3 + (h - 1)], 0)
                gemm(ccw_comm.at[src_slot],
                     ids_ref[3 + (N_HOP - 1) + (h - 1)], 0)

            if h >= 1:
                for i in range(N_SUB):
                    prev_cw[i].wait_send()
                    prev_ccw[i].wait_send()
                if 1 <= h - 1 <= N_HOP - N_SLOT:
                    pl.semaphore_signal(
                        cw_credit, inc=1,
                        device_id=(left,), device_id_type=pl.DeviceIdType.MESH,
                    )
                    pl.semaphore_signal(
                        ccw_credit, inc=1,
                        device_id=(right,), device_id_type=pl.DeviceIdType.MESH,
                    )
            prev_cw, prev_ccw = cur_cw, cur_ccw

        last_slot = N_HOP % N_SLOT
        origin8 = ids_ref[3 + 2 * (N_HOP - 1)]
        hrows = m_per // 2
        for i in range(N_SUB // 2):
            prev_cw[i].wait_recv()
        gemm(cw_comm.at[last_slot, pl.ds(0, hrows)], origin8, 0)
        for i in range(N_SUB // 2, N_SUB):
            prev_ccw[i].wait_recv()
        gemm(ccw_comm.at[last_slot, pl.ds(hrows, hrows)], origin8, hrows)
        for i in range(N_SUB // 2):
            prev_cw[i].wait_send()
        for i in range(N_SUB // 2, N_SUB):
            prev_ccw[i].wait_send()

    return pl.pallas_call(
        body,
        out_shape=jax.ShapeDtypeStruct((N_DEV * m_per, n_per), jnp.float32),
        in_specs=[
            pl.BlockSpec(memory_space=pltpu.SMEM),
            pl.BlockSpec(memory_space=pltpu.VMEM),
            pl.BlockSpec(memory_space=pltpu.VMEM),
        ],
        out_specs=pl.BlockSpec(memory_space=pltpu.VMEM),
        scratch_shapes=[
            pltpu.VMEM((N_SLOT, m_per, k), jnp.float32),
            pltpu.VMEM((N_SLOT, m_per, k), jnp.float32),
            pltpu.SemaphoreType.DMA((N_SLOT, N_SUB)),
            pltpu.SemaphoreType.DMA((N_SLOT, N_SUB)),
            pltpu.SemaphoreType.DMA((N_SLOT, N_SUB)),
            pltpu.SemaphoreType.DMA((N_SLOT, N_SUB)),
            pltpu.SemaphoreType.REGULAR,
            pltpu.SemaphoreType.REGULAR,
        ],
        compiler_params=pltpu.CompilerParams(collective_id=0),
    )(ids, x, w_mat)


# device time: 353203 ns/iter; 2.1675x vs baseline; 1.0413x over previous
import jax
import jax.numpy as jnp
from jax import lax
from jax.experimental import pallas as pl
from jax.experimental.pallas import tpu as pltpu

N_DEV = 16
N_HOP = 8
N_SLOT = 4
N_SUB = 4

RING = [0, 1, 5, 9, 13, 14, 10, 6, 2, 3, 7, 11, 15, 12, 8, 4]
POS = [RING.index(i) for i in range(N_DEV)]


def kernel(x, w_mat):
    m_per, k = x.shape
    k2, n_per = w_mat.shape
    assert k == k2
    sub = m_per // N_SUB

    def body(x_ref, w_ref, out_ref,
             cw_comm, ccw_comm,
             cw_send_sems, cw_recv_sems, ccw_send_sems, ccw_recv_sems,
             cw_credit, ccw_credit):
        my = lax.axis_index("i")

        def tlookup(table, idx):
            acc = jnp.int32(table[0])
            for j in range(1, N_DEV):
                acc = jnp.where(idx == j, jnp.int32(table[j]), acc)
            return acc

        my_pos = tlookup(POS, my)

        def ring_at(offset):
            return tlookup(RING, lax.rem(my_pos + offset + 2 * N_DEV, N_DEV))

        left = ring_at(-1)
        right = ring_at(1)
        cw_origin = [ring_at(-h) for h in range(N_HOP)]
        ccw_origin = [ring_at(h) for h in range(N_HOP)]
        origin8 = ring_at(N_HOP)

        barrier_sem = pltpu.get_barrier_semaphore()
        for nbr in (left, right):
            pl.semaphore_signal(
                barrier_sem, inc=1,
                device_id=(nbr,), device_id_type=pl.DeviceIdType.MESH,
            )
        pl.semaphore_wait(barrier_sem, 2)

        def gemm(rows_ref, origin, row_off):
            out_ref[pl.ds(origin * m_per + row_off, rows_ref.shape[0]), :] = (
                jnp.dot(rows_ref[:, :], w_ref[:, :],
                        preferred_element_type=jnp.float32)
            )

        def make_copy(comm, send_sems, recv_sems, src_slot, dst_slot, i, tgt):
            src = (x_ref if src_slot is None else comm.at[src_slot])
            sem_slot = 0 if src_slot is None else src_slot
            return pltpu.make_async_remote_copy(
                src_ref=src.at[pl.ds(i * sub, sub)],
                dst_ref=comm.at[dst_slot, pl.ds(i * sub, sub)],
                send_sem=send_sems.at[sem_slot, i],
                recv_sem=recv_sems.at[dst_slot, i],
                device_id=(tgt,),
                device_id_type=pl.DeviceIdType.MESH,
            )

        prev_cw = None
        prev_ccw = None
        for h in range(N_HOP):
            src_slot = None if h == 0 else h % N_SLOT
            dst_slot = (h + 1) % N_SLOT
            if h >= N_SLOT:
                pl.semaphore_wait(cw_credit, 1)
                pl.semaphore_wait(ccw_credit, 1)

            if h == N_HOP - 1:
                cw_subs = list(range(N_SUB // 2))
                ccw_subs = list(range(N_SUB // 2, N_SUB))
            else:
                cw_subs = ccw_subs = list(range(N_SUB))
            cur_cw, cur_ccw = {}, {}
            for i in cw_subs:
                if h >= 1:
                    prev_cw[i].wait_recv()
                cur_cw[i] = make_copy(cw_comm, cw_send_sems, cw_recv_sems,
                                      src_slot, dst_slot, i, right)
                cur_cw[i].start()
            for i in ccw_subs:
                if h >= 1:
                    prev_ccw[i].wait_recv()
                cur_ccw[i] = make_copy(ccw_comm, ccw_send_sems, ccw_recv_sems,
                                       src_slot, dst_slot, i, left)
                cur_ccw[i].start()

            if h == 0:
                gemm(x_ref, my, 0)
            else:
                if h == N_HOP - 1:
                    for i in range(N_SUB // 2, N_SUB):
                        prev_cw[i].wait_recv()
                    for i in range(N_SUB // 2):
                        prev_ccw[i].wait_recv()
                gemm(cw_comm.at[src_slot], cw_origin[h], 0)
                gemm(ccw_comm.at[src_slot], ccw_origin[h], 0)

            if h >= 1:
                for i in range(N_SUB):
                    prev_cw[i].wait_send()
                    prev_ccw[i].wait_send()
                if 1 <= h - 1 <= N_HOP - N_SLOT:
                    pl.semaphore_signal(
                        cw_credit, inc=1,
                        device_id=(left,), device_id_type=pl.DeviceIdType.MESH,
                    )
                    pl.semaphore_signal(
                        ccw_credit, inc=1,
                        device_id=(right,), device_id_type=pl.DeviceIdType.MESH,
                    )
            prev_cw, prev_ccw = cur_cw, cur_ccw

        last_slot = N_HOP % N_SLOT
        hrows = m_per // 2
        for i in range(N_SUB // 2):
            prev_cw[i].wait_recv()
        gemm(cw_comm.at[last_slot, pl.ds(0, hrows)], origin8, 0)
        for i in range(N_SUB // 2, N_SUB):
            prev_ccw[i].wait_recv()
        gemm(ccw_comm.at[last_slot, pl.ds(hrows, hrows)], origin8, hrows)
        for i in range(N_SUB // 2):
            prev_cw[i].wait_send()
        for i in range(N_SUB // 2, N_SUB):
            prev_ccw[i].wait_send()

    return pl.pallas_call(
        body,
        out_shape=jax.ShapeDtypeStruct((N_DEV * m_per, n_per), jnp.float32),
        in_specs=[
            pl.BlockSpec(memory_space=pltpu.VMEM),
            pl.BlockSpec(memory_space=pltpu.VMEM),
        ],
        out_specs=pl.BlockSpec(memory_space=pltpu.VMEM),
        scratch_shapes=[
            pltpu.VMEM((N_SLOT, m_per, k), jnp.float32),
            pltpu.VMEM((N_SLOT, m_per, k), jnp.float32),
            pltpu.SemaphoreType.DMA((N_SLOT, N_SUB)),
            pltpu.SemaphoreType.DMA((N_SLOT, N_SUB)),
            pltpu.SemaphoreType.DMA((N_SLOT, N_SUB)),
            pltpu.SemaphoreType.DMA((N_SLOT, N_SUB)),
            pltpu.SemaphoreType.REGULAR,
            pltpu.SemaphoreType.REGULAR,
        ],
        compiler_params=pltpu.CompilerParams(collective_id=0),
    )(x, w_mat)
